# Initial kernel scaffold; baseline (speedup 1.0000x reference)
import jax
import jax.numpy as jnp
from jax.experimental import pallas as pl


def kernel(x, edge_index, batch, W1, b1, W2, b2, Wf1, bf1, Wf2, bf2, Wout, bout):
    raise NotImplementedError("write your pallas kernel here")


# speedup vs baseline: 20.2608x; 20.2608x over previous
import functools

import jax
import jax.numpy as jnp
from jax import lax
from jax.experimental import pallas as pl
from jax.experimental.pallas import tpu as pltpu
from jax.experimental.pallas import tpu_sc as plsc

N = 10000
D = 128
E = 320000

NC = 2
NS = 16
NW = NC * NS
EPW = E // NW
CH = 80
NCH = EPW // CH
NP = 10240
RPT = NP // NS
ZR = 64

BR = 640
GN = NP // BR

_mesh = plsc.VectorSubcoreMesh(core_axis_name="c", subcore_axis_name="s")


def _zero_fill(zb, nrows, width):
    def row(i, _):
        def col(j, _):
            zb[i, pl.ds(j * 16, 16)] = jnp.zeros((16,), jnp.float32)
            return 0

        return lax.fori_loop(0, width // 16, col, 0)

    lax.fori_loop(0, nrows, row, 0)


_DEG_KW = dict(
    out_type=jax.ShapeDtypeStruct((NC, NS, RPT), jnp.float32),
    mesh=_mesh,
    scratch_types=[
        pltpu.VMEM((NCH, CH), jnp.int32),
        pltpu.VMEM((CH,), jnp.int32),
        pltpu.VMEM((CH,), jnp.float32),
        pltpu.VMEM((RPT,), jnp.float32),
        pltpu.VMEM_SHARED((NP,), jnp.float32),
        pltpu.SemaphoreType.DMA,
    ],
)


def _deg_body(dst_hbm, out_hbm, idx_v, db, ones_v, zb_v, acc_sh, dsem):
    cid = lax.axis_index("c")
    sid = lax.axis_index("s")
    wid = sid * NC + cid
    r0 = sid * RPT

    def fill16(i, _):
        ones_v[pl.ds(i * 16, 16)] = jnp.ones((16,), jnp.float32)
        return 0

    lax.fori_loop(0, CH // 16, fill16, 0)

    def zfill(i, _):
        zb_v[pl.ds(i * 16, 16)] = jnp.zeros((16,), jnp.float32)
        return 0

    lax.fori_loop(0, RPT // 16, zfill, 0)
    pltpu.sync_copy(zb_v, acc_sh.at[pl.ds(r0, RPT)])
    pltpu.sync_copy(dst_hbm.at[wid], idx_v)
    plsc.subcore_barrier()

    def chunk(j, _):
        for v in range(CH // 16):
            sl = pl.ds(v * 16, 16)
            db[sl] = idx_v[j, sl]
        pltpu.async_copy(ones_v, acc_sh.at[db], dsem, add=True).wait()
        return 0

    lax.fori_loop(0, NCH, chunk, 0)
    plsc.subcore_barrier()
    pltpu.sync_copy(acc_sh.at[pl.ds(r0, RPT)], zb_v)
    pltpu.sync_copy(zb_v, out_hbm.at[cid, sid])


_deg_kernel = pl.kernel(_deg_body, **_DEG_KW)


_AGG_KW = dict(
    out_type=jax.ShapeDtypeStruct((NC, NP, D), jnp.float32),
    mesh=_mesh,
    scratch_types=[
        pltpu.VMEM((NCH, CH), jnp.int32),
        pltpu.VMEM((CH,), jnp.int32),
        pltpu.VMEM((CH,), jnp.int32),
        pltpu.VMEM((CH, D), jnp.float32),
        pltpu.VMEM((ZR, D), jnp.float32),
        pltpu.VMEM_SHARED((NP, D), jnp.float32),
        pltpu.SemaphoreType.DMA,
        pltpu.SemaphoreType.DMA,
    ],
)


def _agg_body(zs_hbm, pk_hbm, out_hbm, pk, sb, db, rows, zb, acc, sem, sem2):
    cid = lax.axis_index("c")
    sid = lax.axis_index("s")
    wid = sid * NC + cid
    r0 = sid * RPT

    _zero_fill(zb, ZR, D)

    def zcopy(t, _):
        pltpu.sync_copy(zb, acc.at[pl.ds(r0 + t * ZR, ZR)])
        return 0

    lax.fori_loop(0, RPT // ZR, zcopy, 0)
    pltpu.sync_copy(pk_hbm.at[wid], pk)
    plsc.subcore_barrier()

    def chunk(j, _):
        for v in range(CH // 16):
            sl = pl.ds(v * 16, 16)
            code = pk[j, sl]
            sb[sl] = lax.shift_right_logical(code, 14)
            db[sl] = lax.bitwise_and(code, 16383)
        pltpu.async_copy(zs_hbm.at[sb], rows, sem).wait()
        pltpu.async_copy(rows, acc.at[db], sem2, add=True).wait()
        return 0

    lax.fori_loop(0, NCH, chunk, 0)
    plsc.subcore_barrier()

    def rcopy(t, _):
        pltpu.sync_copy(acc.at[pl.ds(r0 + t * ZR, ZR)], zb)
        pltpu.sync_copy(zb, out_hbm.at[cid, pl.ds(r0 + t * ZR, ZR)])
        return 0

    lax.fori_loop(0, RPT // ZR, rcopy, 0)


_agg_kernel = pl.kernel(_agg_body, **_AGG_KW)


def _dinv_block(degp):
    deg = degp[0, :] + degp[1, :] + 1.0
    return lax.rsqrt(deg).reshape(deg.shape[0], 1)


def _lrelu(x):
    return jnp.where(x >= 0, x, 0.01 * x)


def _pre_body(degp_ref, x_ref, w_ref, zs_ref):
    dinv = _dinv_block(degp_ref[...])
    xw = jnp.dot(x_ref[...], w_ref[...], preferred_element_type=jnp.float32)
    zs_ref[...] = xw * dinv


_pre = pl.pallas_call(
    _pre_body,
    grid=(GN,),
    in_specs=[
        pl.BlockSpec((NC, BR), lambda i: (0, i)),
        pl.BlockSpec((BR, D), lambda i: (i, 0)),
        pl.BlockSpec((D, D), lambda i: (0, 0)),
    ],
    out_specs=pl.BlockSpec((BR, D), lambda i: (i, 0)),
    out_shape=jax.ShapeDtypeStruct((NP, D), jnp.float32),
)


def _mid_body(degp_ref, agg_ref, zs_ref, b_ref, w_ref, out_ref):
    dinv = _dinv_block(degp_ref[...])
    h = _lrelu(dinv * (agg_ref[0] + agg_ref[1] + zs_ref[...]) + b_ref[...])
    out_ref[...] = jnp.dot(
        h, w_ref[...], preferred_element_type=jnp.float32) * dinv


_mid = pl.pallas_call(
    _mid_body,
    grid=(GN,),
    in_specs=[
        pl.BlockSpec((NC, BR), lambda i: (0, i)),
        pl.BlockSpec((NC, BR, D), lambda i: (0, i, 0)),
        pl.BlockSpec((BR, D), lambda i: (i, 0)),
        pl.BlockSpec((1, D), lambda i: (0, 0)),
        pl.BlockSpec((D, D), lambda i: (0, 0)),
    ],
    out_specs=pl.BlockSpec((BR, D), lambda i: (i, 0)),
    out_shape=jax.ShapeDtypeStruct((NP, D), jnp.float32),
)


def _post_body(degp_ref, agg_ref, zs_ref, b_ref, wf1_ref, bf1_ref,
               wf2_ref, bf2_ref, wo_ref, bo_ref, out_ref, accs):
    i = pl.program_id(0)
    dinv = _dinv_block(degp_ref[...])
    h = _lrelu(dinv * (agg_ref[0] + agg_ref[1] + zs_ref[...]) + b_ref[...])
    rowid = lax.broadcasted_iota(jnp.int32, (BR, 1), 0) + i * BR
    h = jnp.where(rowid < N, h, 0.0)
    psum = jnp.sum(h, axis=0, keepdims=True)

    @pl.when(i == 0)
    def _():
        accs[...] = psum

    @pl.when(i > 0)
    def _():
        accs[...] = accs[...] + psum

    @pl.when(i == GN - 1)
    def _():
        pooled = accs[...] / float(N)
        f = jnp.dot(pooled, wf1_ref[...],
                    preferred_element_type=jnp.float32) + bf1_ref[...]
        f = _lrelu(f)
        f = jnp.dot(f, wf2_ref[...],
                    preferred_element_type=jnp.float32) + bf2_ref[...]
        f = _lrelu(f)
        out_ref[...] = jnp.dot(
            f, wo_ref[...], preferred_element_type=jnp.float32) + bo_ref[...]


_post = pl.pallas_call(
    _post_body,
    grid=(GN,),
    in_specs=[
        pl.BlockSpec((NC, BR), lambda i: (0, i)),
        pl.BlockSpec((NC, BR, D), lambda i: (0, i, 0)),
        pl.BlockSpec((BR, D), lambda i: (i, 0)),
        pl.BlockSpec((1, D), lambda i: (0, 0)),
        pl.BlockSpec((D, D), lambda i: (0, 0)),
        pl.BlockSpec((1, D), lambda i: (0, 0)),
        pl.BlockSpec((D, D), lambda i: (0, 0)),
        pl.BlockSpec((1, D), lambda i: (0, 0)),
        pl.BlockSpec((D, 1), lambda i: (0, 0)),
        pl.BlockSpec((1, 1), lambda i: (0, 0)),
    ],
    out_specs=pl.BlockSpec((1, 1), lambda i: (0, 0)),
    out_shape=jax.ShapeDtypeStruct((1, 1), jnp.float32),
    scratch_shapes=[pltpu.VMEM((1, D), jnp.float32)],
)


def kernel(x, edge_index, batch, W1, b1, W2, b2, Wf1, bf1, Wf2, bf2, Wout, bout):
    src = edge_index[0]
    dst = edge_index[1]
    pk = jnp.bitwise_or(jnp.left_shift(src, 14), dst).reshape(NW, NCH, CH)
    dst3 = dst.reshape(NW, NCH, CH)
    xpad = jnp.concatenate(
        [x, jnp.zeros((NP - N, D), jnp.float32)], axis=0)
    degp = _deg_kernel(dst3).reshape(NC, NP)
    zs1 = _pre(degp, xpad, W1)
    agg1 = _agg_kernel(zs1, pk)
    zs2 = _mid(degp, agg1, zs1, b1.reshape(1, D), W2)
    agg2 = _agg_kernel(zs2, pk)
    return _post(degp, agg2, zs2, b2.reshape(1, D),
                 Wf1, bf1.reshape(1, D), Wf2, bf2.reshape(1, D),
                 Wout, bout.reshape(1, 1))


# speedup vs baseline: 31.0726x; 1.5336x over previous
import functools

import jax
import jax.numpy as jnp
from jax import lax
from jax.experimental import pallas as pl
from jax.experimental.pallas import tpu as pltpu
from jax.experimental.pallas import tpu_sc as plsc

N = 10000
D = 128
E = 320000

NC = 2
NS = 16
NW = NC * NS
EPW = E // NW
CH = 80
NCH = EPW // CH
NP = 10240
RPT = NP // NS
ZR = 32

BR = 640
GN = NP // BR

_mesh = plsc.VectorSubcoreMesh(core_axis_name="c", subcore_axis_name="s")


def _zero_fill(zb, nrows, width):
    def row(i, _):
        def col(j, _):
            zb[i, pl.ds(j * 16, 16)] = jnp.zeros((16,), jnp.float32)
            return 0

        return lax.fori_loop(0, width // 16, col, 0)

    lax.fori_loop(0, nrows, row, 0)


_DEG_KW = dict(
    out_type=jax.ShapeDtypeStruct((NC, NS, RPT), jnp.float32),
    mesh=_mesh,
    scratch_types=[
        pltpu.VMEM((NCH, CH), jnp.int32),
        pltpu.VMEM((CH,), jnp.int32),
        pltpu.VMEM((CH,), jnp.float32),
        pltpu.VMEM((RPT,), jnp.float32),
        pltpu.VMEM_SHARED((NP,), jnp.float32),
        pltpu.SemaphoreType.DMA,
    ],
)


def _deg_body(dst_hbm, out_hbm, idx_v, db, ones_v, zb_v, acc_sh, dsem):
    cid = lax.axis_index("c")
    sid = lax.axis_index("s")
    wid = sid * NC + cid
    r0 = sid * RPT

    def fill16(i, _):
        ones_v[pl.ds(i * 16, 16)] = jnp.ones((16,), jnp.float32)
        return 0

    lax.fori_loop(0, CH // 16, fill16, 0)

    def zfill(i, _):
        zb_v[pl.ds(i * 16, 16)] = jnp.zeros((16,), jnp.float32)
        return 0

    lax.fori_loop(0, RPT // 16, zfill, 0)
    pltpu.sync_copy(zb_v, acc_sh.at[pl.ds(r0, RPT)])
    pltpu.sync_copy(dst_hbm.at[wid], idx_v)
    plsc.subcore_barrier()

    def chunk(j, _):
        for v in range(CH // 16):
            sl = pl.ds(v * 16, 16)
            db[sl] = idx_v[j, sl]
        pltpu.async_copy(ones_v, acc_sh.at[db], dsem, add=True).wait()
        return 0

    lax.fori_loop(0, NCH, chunk, 0)
    plsc.subcore_barrier()
    pltpu.sync_copy(acc_sh.at[pl.ds(r0, RPT)], zb_v)
    pltpu.sync_copy(zb_v, out_hbm.at[cid, sid])


_deg_kernel = pl.kernel(_deg_body, **_DEG_KW)


_AGG_KW = dict(
    out_type=jax.ShapeDtypeStruct((NC, NP, D), jnp.float32),
    mesh=_mesh,
    scratch_types=[
        pltpu.VMEM((NCH, CH), jnp.int32),
        pltpu.VMEM((CH,), jnp.int32),
        pltpu.VMEM((CH,), jnp.int32),
        pltpu.VMEM((CH,), jnp.int32),
        pltpu.VMEM((CH,), jnp.int32),
        pltpu.VMEM((CH, D), jnp.float32),
        pltpu.VMEM((CH, D), jnp.float32),
        pltpu.VMEM((ZR, D), jnp.float32),
        pltpu.VMEM_SHARED((NP, D), jnp.float32),
        pltpu.SemaphoreType.DMA,
        pltpu.SemaphoreType.DMA,
        pltpu.SemaphoreType.DMA,
    ],
)


def _agg_body(zs_hbm, pk_hbm, out_hbm, pk, sb0, db0, sb1, db1,
              rows0, rows1, zb, acc, sem0, sem1, sem2):
    cid = lax.axis_index("c")
    sid = lax.axis_index("s")
    wid = sid * NC + cid
    r0 = sid * RPT

    _zero_fill(zb, ZR, D)

    def zcopy(t, _):
        pltpu.sync_copy(zb, acc.at[pl.ds(r0 + t * ZR, ZR)])
        return 0

    lax.fori_loop(0, RPT // ZR, zcopy, 0)
    pltpu.sync_copy(pk_hbm.at[wid], pk)
    plsc.subcore_barrier()

    def decode(j, sb, db):
        for v in range(CH // 16):
            sl = pl.ds(v * 16, 16)
            code = pk[j, sl]
            sb[sl] = lax.shift_right_logical(code, 14)
            db[sl] = lax.bitwise_and(code, 16383)

    decode(0, sb0, db0)
    pltpu.async_copy(zs_hbm.at[sb0], rows0, sem0)

    def pair(jj, _):
        i0 = 2 * jj
        decode(i0 + 1, sb1, db1)
        pltpu.async_copy(zs_hbm.at[sb1], rows1, sem1)
        pltpu.make_async_copy(zs_hbm.at[sb0], rows0, sem0).wait()
        pltpu.async_copy(rows0, acc.at[db0], sem2, add=True).wait()
        decode(i0 + 2, sb0, db0)
        pltpu.async_copy(zs_hbm.at[sb0], rows0, sem0)
        pltpu.make_async_copy(zs_hbm.at[sb1], rows1, sem1).wait()
        pltpu.async_copy(rows1, acc.at[db1], sem2, add=True).wait()
        return 0

    lax.fori_loop(0, NCH // 2, pair, 0)
    pltpu.make_async_copy(zs_hbm.at[sb0], rows0, sem0).wait()
    pltpu.async_copy(rows0, acc.at[db0], sem2, add=True).wait()
    plsc.subcore_barrier()

    def rcopy(t, _):
        pltpu.sync_copy(acc.at[pl.ds(r0 + t * ZR, ZR)], zb)
        pltpu.sync_copy(zb, out_hbm.at[cid, pl.ds(r0 + t * ZR, ZR)])
        return 0

    lax.fori_loop(0, RPT // ZR, rcopy, 0)


_agg_kernel = pl.kernel(_agg_body, **_AGG_KW)


def _dinv_block(degp):
    deg = degp[0, :] + degp[1, :] + 1.0
    return lax.rsqrt(deg).reshape(deg.shape[0], 1)


def _lrelu(x):
    return jnp.where(x >= 0, x, 0.01 * x)


def _pre_body(degp_ref, x_ref, w_ref, zs_ref):
    dinv = _dinv_block(degp_ref[...])
    xw = jnp.dot(x_ref[...], w_ref[...], preferred_element_type=jnp.float32)
    zs_ref[...] = xw * dinv


_pre = pl.pallas_call(
    _pre_body,
    grid=(GN,),
    in_specs=[
        pl.BlockSpec((NC, BR), lambda i: (0, i)),
        pl.BlockSpec((BR, D), lambda i: (i, 0)),
        pl.BlockSpec((D, D), lambda i: (0, 0)),
    ],
    out_specs=pl.BlockSpec((BR, D), lambda i: (i, 0)),
    out_shape=jax.ShapeDtypeStruct((NP, D), jnp.float32),
)


def _mid_body(degp_ref, agg_ref, zs_ref, b_ref, w_ref, out_ref):
    dinv = _dinv_block(degp_ref[...])
    h = _lrelu(dinv * (agg_ref[0] + agg_ref[1] + zs_ref[...]) + b_ref[...])
    out_ref[...] = jnp.dot(
        h, w_ref[...], preferred_element_type=jnp.float32) * dinv


_mid = pl.pallas_call(
    _mid_body,
    grid=(GN,),
    in_specs=[
        pl.BlockSpec((NC, BR), lambda i: (0, i)),
        pl.BlockSpec((NC, BR, D), lambda i: (0, i, 0)),
        pl.BlockSpec((BR, D), lambda i: (i, 0)),
        pl.BlockSpec((1, D), lambda i: (0, 0)),
        pl.BlockSpec((D, D), lambda i: (0, 0)),
    ],
    out_specs=pl.BlockSpec((BR, D), lambda i: (i, 0)),
    out_shape=jax.ShapeDtypeStruct((NP, D), jnp.float32),
)


def _post_body(degp_ref, agg_ref, zs_ref, b_ref, wf1_ref, bf1_ref,
               wf2_ref, bf2_ref, wo_ref, bo_ref, out_ref, accs):
    i = pl.program_id(0)
    dinv = _dinv_block(degp_ref[...])
    h = _lrelu(dinv * (agg_ref[0] + agg_ref[1] + zs_ref[...]) + b_ref[...])
    rowid = lax.broadcasted_iota(jnp.int32, (BR, 1), 0) + i * BR
    h = jnp.where(rowid < N, h, 0.0)
    psum = jnp.sum(h, axis=0, keepdims=True)

    @pl.when(i == 0)
    def _():
        accs[...] = psum

    @pl.when(i > 0)
    def _():
        accs[...] = accs[...] + psum

    @pl.when(i == GN - 1)
    def _():
        pooled = accs[...] / float(N)
        f = jnp.dot(pooled, wf1_ref[...],
                    preferred_element_type=jnp.float32) + bf1_ref[...]
        f = _lrelu(f)
        f = jnp.dot(f, wf2_ref[...],
                    preferred_element_type=jnp.float32) + bf2_ref[...]
        f = _lrelu(f)
        out_ref[...] = jnp.dot(
            f, wo_ref[...], preferred_element_type=jnp.float32) + bo_ref[...]


_post = pl.pallas_call(
    _post_body,
    grid=(GN,),
    in_specs=[
        pl.BlockSpec((NC, BR), lambda i: (0, i)),
        pl.BlockSpec((NC, BR, D), lambda i: (0, i, 0)),
        pl.BlockSpec((BR, D), lambda i: (i, 0)),
        pl.BlockSpec((1, D), lambda i: (0, 0)),
        pl.BlockSpec((D, D), lambda i: (0, 0)),
        pl.BlockSpec((1, D), lambda i: (0, 0)),
        pl.BlockSpec((D, D), lambda i: (0, 0)),
        pl.BlockSpec((1, D), lambda i: (0, 0)),
        pl.BlockSpec((D, 1), lambda i: (0, 0)),
        pl.BlockSpec((1, 1), lambda i: (0, 0)),
    ],
    out_specs=pl.BlockSpec((1, 1), lambda i: (0, 0)),
    out_shape=jax.ShapeDtypeStruct((1, 1), jnp.float32),
    scratch_shapes=[pltpu.VMEM((1, D), jnp.float32)],
)


def kernel(x, edge_index, batch, W1, b1, W2, b2, Wf1, bf1, Wf2, bf2, Wout, bout):
    src = edge_index[0]
    dst = edge_index[1]
    pk = jnp.bitwise_or(jnp.left_shift(src, 14), dst).reshape(NW, NCH, CH)
    dst3 = dst.reshape(NW, NCH, CH)
    xpad = jnp.concatenate(
        [x, jnp.zeros((NP - N, D), jnp.float32)], axis=0)
    degp = _deg_kernel(dst3).reshape(NC, NP)
    zs1 = _pre(degp, xpad, W1)
    agg1 = _agg_kernel(zs1, pk)
    zs2 = _mid(degp, agg1, zs1, b1.reshape(1, D), W2)
    agg2 = _agg_kernel(zs2, pk)
    return _post(degp, agg2, zs2, b2.reshape(1, D),
                 Wf1, bf1.reshape(1, D), Wf2, bf2.reshape(1, D),
                 Wout, bout.reshape(1, 1))


# speedup vs baseline: 31.7192x; 1.0208x over previous
import functools

import jax
import jax.numpy as jnp
from jax import lax
from jax.experimental import pallas as pl
from jax.experimental.pallas import tpu as pltpu
from jax.experimental.pallas import tpu_sc as plsc

N = 10000
D = 128
E = 320000

NC = 2
NS = 16
NW = NC * NS
EPW = E // NW
CH = 80
NCH = EPW // CH
NP = 10240
RPT = NP // NS
ZR = 32

BR = 640
GN = NP // BR

_mesh = plsc.VectorSubcoreMesh(core_axis_name="c", subcore_axis_name="s")


def _zero_fill(zb, nrows, width):
    def row(i, _):
        def col(j, _):
            zb[i, pl.ds(j * 16, 16)] = jnp.zeros((16,), jnp.float32)
            return 0

        return lax.fori_loop(0, width // 16, col, 0)

    lax.fori_loop(0, nrows, row, 0)


_DEG_KW = dict(
    out_type=jax.ShapeDtypeStruct((NC, NS, RPT), jnp.float32),
    mesh=_mesh,
    scratch_types=[
        pltpu.VMEM((NCH, CH), jnp.int32),
        pltpu.VMEM((CH,), jnp.int32),
        pltpu.VMEM((CH,), jnp.int32),
        pltpu.VMEM((CH,), jnp.float32),
        pltpu.VMEM((RPT,), jnp.float32),
        pltpu.VMEM_SHARED((NP,), jnp.float32),
        pltpu.SemaphoreType.DMA,
        pltpu.SemaphoreType.DMA,
    ],
)


def _deg_body(dst_hbm, out_hbm, idx_v, db0, db1, ones_v, zb_v, acc_sh,
              dsem0, dsem1):
    cid = lax.axis_index("c")
    sid = lax.axis_index("s")
    wid = sid * NC + cid
    r0 = sid * RPT

    def fill16(i, _):
        ones_v[pl.ds(i * 16, 16)] = jnp.ones((16,), jnp.float32)
        return 0

    lax.fori_loop(0, CH // 16, fill16, 0)

    def zfill(i, _):
        zb_v[pl.ds(i * 16, 16)] = jnp.zeros((16,), jnp.float32)
        return 0

    lax.fori_loop(0, RPT // 16, zfill, 0)
    pltpu.sync_copy(zb_v, acc_sh.at[pl.ds(r0, RPT)])
    pltpu.sync_copy(dst_hbm.at[wid], idx_v)
    plsc.subcore_barrier()

    def decode(j, db):
        for v in range(CH // 16):
            sl = pl.ds(v * 16, 16)
            db[sl] = idx_v[j, sl]

    decode(0, db0)
    pltpu.async_copy(ones_v, acc_sh.at[db0], dsem0, add=True)

    def pair(jj, _):
        i0 = 2 * jj
        decode(i0 + 1, db1)
        pltpu.async_copy(ones_v, acc_sh.at[db1], dsem1, add=True)
        pltpu.make_async_copy(ones_v, acc_sh.at[db0], dsem0).wait()
        decode(i0 + 2, db0)
        pltpu.async_copy(ones_v, acc_sh.at[db0], dsem0, add=True)
        pltpu.make_async_copy(ones_v, acc_sh.at[db1], dsem1).wait()
        return 0

    lax.fori_loop(0, NCH // 2, pair, 0)
    pltpu.make_async_copy(ones_v, acc_sh.at[db0], dsem0).wait()
    plsc.subcore_barrier()
    pltpu.sync_copy(acc_sh.at[pl.ds(r0, RPT)], zb_v)
    pltpu.sync_copy(zb_v, out_hbm.at[cid, sid])


_deg_kernel = pl.kernel(_deg_body, **_DEG_KW)


_AGG_KW = dict(
    out_type=jax.ShapeDtypeStruct((NC, NP, D), jnp.float32),
    mesh=_mesh,
    scratch_types=[
        pltpu.VMEM((NCH, CH), jnp.int32),
        pltpu.VMEM((CH,), jnp.int32),
        pltpu.VMEM((CH,), jnp.int32),
        pltpu.VMEM((CH,), jnp.int32),
        pltpu.VMEM((CH,), jnp.int32),
        pltpu.VMEM((CH, D), jnp.float32),
        pltpu.VMEM((CH, D), jnp.float32),
        pltpu.VMEM((ZR, D), jnp.float32),
        pltpu.VMEM_SHARED((NP, D), jnp.float32),
        pltpu.SemaphoreType.DMA,
        pltpu.SemaphoreType.DMA,
        pltpu.SemaphoreType.DMA,
    ],
)


def _agg_body(zs_hbm, pk_hbm, out_hbm, pk, sb0, db0, sb1, db1,
              rows0, rows1, zb, acc, sem0, sem1, sem2):
    cid = lax.axis_index("c")
    sid = lax.axis_index("s")
    wid = sid * NC + cid
    r0 = sid * RPT

    _zero_fill(zb, ZR, D)

    def zcopy(t, _):
        pltpu.sync_copy(zb, acc.at[pl.ds(r0 + t * ZR, ZR)])
        return 0

    lax.fori_loop(0, RPT // ZR, zcopy, 0)
    pltpu.sync_copy(pk_hbm.at[wid], pk)
    plsc.subcore_barrier()

    def decode(j, sb, db):
        for v in range(CH // 16):
            sl = pl.ds(v * 16, 16)
            code = pk[j, sl]
            sb[sl] = lax.shift_right_logical(code, 14)
            db[sl] = lax.bitwise_and(code, 16383)

    decode(0, sb0, db0)
    pltpu.async_copy(zs_hbm.at[sb0], rows0, sem0)

    def pair(jj, _):
        i0 = 2 * jj
        decode(i0 + 1, sb1, db1)
        pltpu.async_copy(zs_hbm.at[sb1], rows1, sem1)
        pltpu.make_async_copy(zs_hbm.at[sb0], rows0, sem0).wait()
        pltpu.async_copy(rows0, acc.at[db0], sem2, add=True).wait()
        decode(i0 + 2, sb0, db0)
        pltpu.async_copy(zs_hbm.at[sb0], rows0, sem0)
        pltpu.make_async_copy(zs_hbm.at[sb1], rows1, sem1).wait()
        pltpu.async_copy(rows1, acc.at[db1], sem2, add=True).wait()
        return 0

    lax.fori_loop(0, NCH // 2, pair, 0)
    pltpu.make_async_copy(zs_hbm.at[sb0], rows0, sem0).wait()
    pltpu.async_copy(rows0, acc.at[db0], sem2, add=True).wait()
    plsc.subcore_barrier()

    def rcopy(t, _):
        pltpu.sync_copy(acc.at[pl.ds(r0 + t * ZR, ZR)], zb)
        pltpu.sync_copy(zb, out_hbm.at[cid, pl.ds(r0 + t * ZR, ZR)])
        return 0

    lax.fori_loop(0, RPT // ZR, rcopy, 0)


_agg_kernel = pl.kernel(_agg_body, **_AGG_KW)


def _dinv_block(degp):
    deg = degp[0, :] + degp[1, :] + 1.0
    return lax.rsqrt(deg).reshape(deg.shape[0], 1)


def _lrelu(x):
    return jnp.where(x >= 0, x, 0.01 * x)


def _pre_body(degp_ref, x_ref, w_ref, zs_ref):
    dinv = _dinv_block(degp_ref[...])
    xw = jnp.dot(x_ref[...], w_ref[...], preferred_element_type=jnp.float32)
    zs_ref[...] = xw * dinv


_pre = pl.pallas_call(
    _pre_body,
    grid=(GN,),
    in_specs=[
        pl.BlockSpec((NC, BR), lambda i: (0, i)),
        pl.BlockSpec((BR, D), lambda i: (i, 0)),
        pl.BlockSpec((D, D), lambda i: (0, 0)),
    ],
    out_specs=pl.BlockSpec((BR, D), lambda i: (i, 0)),
    out_shape=jax.ShapeDtypeStruct((NP, D), jnp.float32),
)


def _mid_body(degp_ref, agg_ref, zs_ref, b_ref, w_ref, out_ref):
    dinv = _dinv_block(degp_ref[...])
    h = _lrelu(dinv * (agg_ref[0] + agg_ref[1] + zs_ref[...]) + b_ref[...])
    out_ref[...] = jnp.dot(
        h, w_ref[...], preferred_element_type=jnp.float32) * dinv


_mid = pl.pallas_call(
    _mid_body,
    grid=(GN,),
    in_specs=[
        pl.BlockSpec((NC, BR), lambda i: (0, i)),
        pl.BlockSpec((NC, BR, D), lambda i: (0, i, 0)),
        pl.BlockSpec((BR, D), lambda i: (i, 0)),
        pl.BlockSpec((1, D), lambda i: (0, 0)),
        pl.BlockSpec((D, D), lambda i: (0, 0)),
    ],
    out_specs=pl.BlockSpec((BR, D), lambda i: (i, 0)),
    out_shape=jax.ShapeDtypeStruct((NP, D), jnp.float32),
)


def _post_body(degp_ref, agg_ref, zs_ref, b_ref, wf1_ref, bf1_ref,
               wf2_ref, bf2_ref, wo_ref, bo_ref, out_ref, accs):
    i = pl.program_id(0)
    dinv = _dinv_block(degp_ref[...])
    h = _lrelu(dinv * (agg_ref[0] + agg_ref[1] + zs_ref[...]) + b_ref[...])
    rowid = lax.broadcasted_iota(jnp.int32, (BR, 1), 0) + i * BR
    h = jnp.where(rowid < N, h, 0.0)
    psum = jnp.sum(h, axis=0, keepdims=True)

    @pl.when(i == 0)
    def _():
        accs[...] = psum

    @pl.when(i > 0)
    def _():
        accs[...] = accs[...] + psum

    @pl.when(i == GN - 1)
    def _():
        pooled = accs[...] / float(N)
        f = jnp.dot(pooled, wf1_ref[...],
                    preferred_element_type=jnp.float32) + bf1_ref[...]
        f = _lrelu(f)
        f = jnp.dot(f, wf2_ref[...],
                    preferred_element_type=jnp.float32) + bf2_ref[...]
        f = _lrelu(f)
        out_ref[...] = jnp.dot(
            f, wo_ref[...], preferred_element_type=jnp.float32) + bo_ref[...]


_post = pl.pallas_call(
    _post_body,
    grid=(GN,),
    in_specs=[
        pl.BlockSpec((NC, BR), lambda i: (0, i)),
        pl.BlockSpec((NC, BR, D), lambda i: (0, i, 0)),
        pl.BlockSpec((BR, D), lambda i: (i, 0)),
        pl.BlockSpec((1, D), lambda i: (0, 0)),
        pl.BlockSpec((D, D), lambda i: (0, 0)),
        pl.BlockSpec((1, D), lambda i: (0, 0)),
        pl.BlockSpec((D, D), lambda i: (0, 0)),
        pl.BlockSpec((1, D), lambda i: (0, 0)),
        pl.BlockSpec((D, 1), lambda i: (0, 0)),
        pl.BlockSpec((1, 1), lambda i: (0, 0)),
    ],
    out_specs=pl.BlockSpec((1, 1), lambda i: (0, 0)),
    out_shape=jax.ShapeDtypeStruct((1, 1), jnp.float32),
    scratch_shapes=[pltpu.VMEM((1, D), jnp.float32)],
)


def kernel(x, edge_index, batch, W1, b1, W2, b2, Wf1, bf1, Wf2, bf2, Wout, bout):
    src = edge_index[0]
    dst = edge_index[1]
    pk = jnp.bitwise_or(jnp.left_shift(src, 14), dst).reshape(NW, NCH, CH)
    dst3 = dst.reshape(NW, NCH, CH)
    xpad = jnp.concatenate(
        [x, jnp.zeros((NP - N, D), jnp.float32)], axis=0)
    degp = _deg_kernel(dst3).reshape(NC, NP)
    zs1 = _pre(degp, xpad, W1)
    agg1 = _agg_kernel(zs1, pk)
    zs2 = _mid(degp, agg1, zs1, b1.reshape(1, D), W2)
    agg2 = _agg_kernel(zs2, pk)
    return _post(degp, agg2, zs2, b2.reshape(1, D),
                 Wf1, bf1.reshape(1, D), Wf2, bf2.reshape(1, D),
                 Wout, bout.reshape(1, 1))


# speedup vs baseline: 31.7212x; 1.0001x over previous
import jax
import jax.numpy as jnp
from jax import lax
from jax.experimental import pallas as pl
from jax.experimental.pallas import tpu as pltpu
from jax.experimental.pallas import tpu_sc as plsc

N = 10000
D = 128
E = 320000

NC = 2
NS = 16
NW = NC * NS
EPW = E // NW
CH = 80
NCH = EPW // CH
NP = 10240
RPT = NP // NS
ZR = 32

BR = 640
GN = NP // BR

_mesh = plsc.VectorSubcoreMesh(core_axis_name="c", subcore_axis_name="s")


def _zero_fill(zb, nrows, width):
    def row(i, _):
        def col(j, _):
            zb[i, pl.ds(j * 16, 16)] = jnp.zeros((16,), jnp.float32)
            return 0

        return lax.fori_loop(0, width // 16, col, 0)

    lax.fori_loop(0, nrows, row, 0)


_DEG_KW = dict(
    out_type=jax.ShapeDtypeStruct((NC, NS, RPT), jnp.float32),
    mesh=_mesh,
    scratch_types=[
        pltpu.VMEM((NCH, CH), jnp.int32),
        pltpu.VMEM((CH,), jnp.int32),
        pltpu.VMEM((CH,), jnp.int32),
        pltpu.VMEM((CH,), jnp.float32),
        pltpu.VMEM((RPT,), jnp.float32),
        pltpu.VMEM_SHARED((NP,), jnp.float32),
        pltpu.SemaphoreType.DMA,
        pltpu.SemaphoreType.DMA,
    ],
)


def _deg_body(dst_hbm, out_hbm, idx_v, db0, db1, ones_v, zb_v, acc_sh,
              dsem0, dsem1):
    cid = lax.axis_index("c")
    sid = lax.axis_index("s")
    wid = sid * NC + cid
    r0 = sid * RPT

    def fill16(i, _):
        ones_v[pl.ds(i * 16, 16)] = jnp.ones((16,), jnp.float32)
        return 0

    lax.fori_loop(0, CH // 16, fill16, 0)

    def zfill(i, _):
        zb_v[pl.ds(i * 16, 16)] = jnp.zeros((16,), jnp.float32)
        return 0

    lax.fori_loop(0, RPT // 16, zfill, 0)
    pltpu.sync_copy(zb_v, acc_sh.at[pl.ds(r0, RPT)])
    pltpu.sync_copy(dst_hbm.at[wid], idx_v)
    plsc.subcore_barrier()

    def decode(j, db):
        for v in range(CH // 16):
            sl = pl.ds(v * 16, 16)
            db[sl] = idx_v[j, sl]

    decode(0, db0)
    pltpu.async_copy(ones_v, acc_sh.at[db0], dsem0, add=True)

    def pair(jj, _):
        i0 = 2 * jj
        decode(i0 + 1, db1)
        pltpu.async_copy(ones_v, acc_sh.at[db1], dsem1, add=True)
        pltpu.make_async_copy(ones_v, acc_sh.at[db0], dsem0).wait()
        decode(i0 + 2, db0)
        pltpu.async_copy(ones_v, acc_sh.at[db0], dsem0, add=True)
        pltpu.make_async_copy(ones_v, acc_sh.at[db1], dsem1).wait()
        return 0

    lax.fori_loop(0, NCH // 2, pair, 0)
    pltpu.make_async_copy(ones_v, acc_sh.at[db0], dsem0).wait()
    plsc.subcore_barrier()
    pltpu.sync_copy(acc_sh.at[pl.ds(r0, RPT)], zb_v)
    pltpu.sync_copy(zb_v, out_hbm.at[cid, sid])


_deg_kernel = pl.kernel(_deg_body, **_DEG_KW)


_AGG_KW = dict(
    out_type=jax.ShapeDtypeStruct((NC, NP, D), jnp.float32),
    mesh=_mesh,
    scratch_types=[
        pltpu.VMEM((NCH, CH), jnp.int32),
        pltpu.VMEM((CH,), jnp.int32),
        pltpu.VMEM((CH,), jnp.int32),
        pltpu.VMEM((CH,), jnp.int32),
        pltpu.VMEM((CH,), jnp.int32),
        pltpu.VMEM((CH, D), jnp.float32),
        pltpu.VMEM((CH, D), jnp.float32),
        pltpu.VMEM((ZR, D), jnp.float32),
        pltpu.VMEM_SHARED((NP, D), jnp.float32),
        pltpu.SemaphoreType.DMA,
        pltpu.SemaphoreType.DMA,
        pltpu.SemaphoreType.DMA,
    ],
)


def _agg_body(zs_hbm, pk_hbm, out_hbm, pk, sb0, db0, sb1, db1,
              rows0, rows1, zb, acc, sem0, sem1, sem2):
    cid = lax.axis_index("c")
    sid = lax.axis_index("s")
    wid = sid * NC + cid
    r0 = sid * RPT

    _zero_fill(zb, ZR, D)

    def zcopy(t, _):
        pltpu.sync_copy(zb, acc.at[pl.ds(r0 + t * ZR, ZR)])
        return 0

    lax.fori_loop(0, RPT // ZR, zcopy, 0)
    pltpu.sync_copy(pk_hbm.at[wid], pk)
    plsc.subcore_barrier()

    def decode(j, sb, db):
        for v in range(CH // 16):
            sl = pl.ds(v * 16, 16)
            code = pk[j, sl]
            sb[sl] = lax.shift_right_logical(code, 14)
            db[sl] = lax.bitwise_and(code, 16383)

    decode(0, sb0, db0)
    pltpu.async_copy(zs_hbm.at[sb0], rows0, sem0)

    def pair(jj, _):
        i0 = 2 * jj
        decode(i0 + 1, sb1, db1)
        pltpu.async_copy(zs_hbm.at[sb1], rows1, sem1)
        pltpu.make_async_copy(zs_hbm.at[sb0], rows0, sem0).wait()
        pltpu.async_copy(rows0, acc.at[db0], sem2, add=True).wait()
        decode(i0 + 2, sb0, db0)
        pltpu.async_copy(zs_hbm.at[sb0], rows0, sem0)
        pltpu.make_async_copy(zs_hbm.at[sb1], rows1, sem1).wait()
        pltpu.async_copy(rows1, acc.at[db1], sem2, add=True).wait()
        return 0

    lax.fori_loop(0, NCH // 2, pair, 0)
    pltpu.make_async_copy(zs_hbm.at[sb0], rows0, sem0).wait()
    pltpu.async_copy(rows0, acc.at[db0], sem2, add=True).wait()
    plsc.subcore_barrier()

    def rcopy(t, _):
        pltpu.sync_copy(acc.at[pl.ds(r0 + t * ZR, ZR)], zb)
        pltpu.sync_copy(zb, out_hbm.at[cid, pl.ds(r0 + t * ZR, ZR)])
        return 0

    lax.fori_loop(0, RPT // ZR, rcopy, 0)


_agg_kernel = pl.kernel(_agg_body, **_AGG_KW)


def _dinv_block(degp):
    deg = degp[0, :] + degp[1, :] + 1.0
    return lax.rsqrt(deg).reshape(deg.shape[0], 1)


def _lrelu(x):
    return jnp.where(x >= 0, x, 0.01 * x)


def _pre_body(degp_ref, x_ref, w_ref, zs_ref):
    dinv = _dinv_block(degp_ref[...])
    xw = jnp.dot(x_ref[...], w_ref[...], preferred_element_type=jnp.float32)
    zs_ref[...] = xw * dinv


_pre = pl.pallas_call(
    _pre_body,
    grid=(GN,),
    in_specs=[
        pl.BlockSpec((NC, BR), lambda i: (0, i)),
        pl.BlockSpec((BR, D), lambda i: (i, 0)),
        pl.BlockSpec((D, D), lambda i: (0, 0)),
    ],
    out_specs=pl.BlockSpec((BR, D), lambda i: (i, 0)),
    out_shape=jax.ShapeDtypeStruct((NP, D), jnp.float32),
)


def _mid_body(degp_ref, agg_ref, zs_ref, b_ref, w_ref, out_ref):
    dinv = _dinv_block(degp_ref[...])
    h = _lrelu(dinv * (agg_ref[0] + agg_ref[1] + zs_ref[...]) + b_ref[...])
    out_ref[...] = jnp.dot(
        h, w_ref[...], preferred_element_type=jnp.float32) * dinv


_mid = pl.pallas_call(
    _mid_body,
    grid=(GN,),
    in_specs=[
        pl.BlockSpec((NC, BR), lambda i: (0, i)),
        pl.BlockSpec((NC, BR, D), lambda i: (0, i, 0)),
        pl.BlockSpec((BR, D), lambda i: (i, 0)),
        pl.BlockSpec((1, D), lambda i: (0, 0)),
        pl.BlockSpec((D, D), lambda i: (0, 0)),
    ],
    out_specs=pl.BlockSpec((BR, D), lambda i: (i, 0)),
    out_shape=jax.ShapeDtypeStruct((NP, D), jnp.float32),
)


def _post_body(degp_ref, agg_ref, zs_ref, b_ref, wf1_ref, bf1_ref,
               wf2_ref, bf2_ref, wo_ref, bo_ref, out_ref, accs):
    i = pl.program_id(0)
    dinv = _dinv_block(degp_ref[...])
    h = _lrelu(dinv * (agg_ref[0] + agg_ref[1] + zs_ref[...]) + b_ref[...])
    rowid = lax.broadcasted_iota(jnp.int32, (BR, 1), 0) + i * BR
    h = jnp.where(rowid < N, h, 0.0)
    psum = jnp.sum(h, axis=0, keepdims=True)

    @pl.when(i == 0)
    def _():
        accs[...] = psum

    @pl.when(i > 0)
    def _():
        accs[...] = accs[...] + psum

    @pl.when(i == GN - 1)
    def _():
        pooled = accs[...] / float(N)
        f = jnp.dot(pooled, wf1_ref[...],
                    preferred_element_type=jnp.float32) + bf1_ref[...]
        f = _lrelu(f)
        f = jnp.dot(f, wf2_ref[...],
                    preferred_element_type=jnp.float32) + bf2_ref[...]
        f = _lrelu(f)
        out_ref[...] = jnp.dot(
            f, wo_ref[...], preferred_element_type=jnp.float32) + bo_ref[...]


_post = pl.pallas_call(
    _post_body,
    grid=(GN,),
    in_specs=[
        pl.BlockSpec((NC, BR), lambda i: (0, i)),
        pl.BlockSpec((NC, BR, D), lambda i: (0, i, 0)),
        pl.BlockSpec((BR, D), lambda i: (i, 0)),
        pl.BlockSpec((1, D), lambda i: (0, 0)),
        pl.BlockSpec((D, D), lambda i: (0, 0)),
        pl.BlockSpec((1, D), lambda i: (0, 0)),
        pl.BlockSpec((D, D), lambda i: (0, 0)),
        pl.BlockSpec((1, D), lambda i: (0, 0)),
        pl.BlockSpec((D, 1), lambda i: (0, 0)),
        pl.BlockSpec((1, 1), lambda i: (0, 0)),
    ],
    out_specs=pl.BlockSpec((1, 1), lambda i: (0, 0)),
    out_shape=jax.ShapeDtypeStruct((1, 1), jnp.float32),
    scratch_shapes=[pltpu.VMEM((1, D), jnp.float32)],
)


def kernel(x, edge_index, batch, W1, b1, W2, b2, Wf1, bf1, Wf2, bf2, Wout, bout):
    src = edge_index[0]
    dst = edge_index[1]
    pk = jnp.bitwise_or(jnp.left_shift(src, 14), dst).reshape(NW, NCH, CH)
    dst3 = dst.reshape(NW, NCH, CH)
    xpad = jnp.concatenate(
        [x, jnp.zeros((NP - N, D), jnp.float32)], axis=0)
    degp = _deg_kernel(dst3).reshape(NC, NP)
    zs1 = _pre(degp, xpad, W1)
    agg1 = _agg_kernel(zs1, pk)
    zs2 = _mid(degp, agg1, zs1, b1.reshape(1, D), W2)
    agg2 = _agg_kernel(zs2, pk)
    return _post(degp, agg2, zs2, b2.reshape(1, D),
                 Wf1, bf1.reshape(1, D), Wf2, bf2.reshape(1, D),
                 Wout, bout.reshape(1, 1))
